# Initial kernel scaffold; baseline (speedup 1.0000x reference)
import jax
import jax.numpy as jnp
from jax.experimental import pallas as pl


def kernel(x, weight):
    raise NotImplementedError("write your pallas kernel here")


# speedup vs baseline: 1.4923x; 1.4923x over previous
import functools

import jax
import jax.numpy as jnp
from jax import lax
from jax.experimental import pallas as pl
from jax.experimental.pallas import tpu as pltpu
from jax.experimental.pallas import tpu_sc as plsc

VOCAB = 1000000
DIM = 32
B = 4096
L = 200

_N = B * L
_IDX_MINOR = 128
_K = 10
_CHUNK = _K * _IDX_MINOR


def _make_kernel(n_rows):
    info = plsc.get_sparse_core_info()
    nw = info.num_cores * info.num_subcores
    assert n_rows % (nw * _CHUNK * 2) == 0
    ch_per_w = n_rows // (nw * _CHUNK)
    blocks_per_w = ch_per_w * _K

    mesh = plsc.VectorSubcoreMesh(core_axis_name="c", subcore_axis_name="s")

    @functools.partial(
        pl.kernel,
        mesh=mesh,
        out_type=jax.ShapeDtypeStruct((n_rows, DIM), jnp.float32),
        scratch_types=[
            pltpu.VMEM((_K, _IDX_MINOR), jnp.int32),
            pltpu.VMEM((_K, _IDX_MINOR), jnp.int32),
            pltpu.VMEM((_CHUNK, DIM), jnp.float32),
            pltpu.VMEM((_CHUNK, DIM), jnp.float32),
            pltpu.SemaphoreType.DMA,
            pltpu.SemaphoreType.DMA,
            pltpu.SemaphoreType.DMA,
            pltpu.SemaphoreType.DMA,
        ],
        compiler_params=pltpu.CompilerParams(use_tc_tiling_on_sc=False),
    )
    def k(idx_hbm, tbl_hbm, out_hbm, idx0, idx1, rows0, rows1,
          sem_idx, sem_g, sem_o0, sem_o1):
        wid = lax.axis_index("s") * info.num_cores + lax.axis_index("c")
        blk_base = wid * blocks_per_w
        idx_v = (idx0, idx1)
        rows_v = (rows0, rows1)
        sem_o = (sem_o0, sem_o1)

        def idx_slice(g):
            return idx_hbm.at[pl.ds(blk_base + g * _K, _K)]

        def out_slice(g):
            return out_hbm.at[pl.ds((blk_base + g * _K) * _IDX_MINOR, _CHUNK)]

        def step(g, b, wait_store):
            if wait_store:
                pltpu.make_async_copy(rows_v[b], out_slice(g - 2),
                                      sem_o[b]).wait()
            pltpu.make_async_copy(idx_slice(g), idx_v[b], sem_idx).wait()
            gn = jnp.minimum(g + 1, ch_per_w - 1)
            pltpu.async_copy(idx_slice(gn), idx_v[1 - b], sem_idx)
            copies = [
                pltpu.async_copy(
                    tbl_hbm.at[idx_v[b].at[j]],
                    rows_v[b].at[pl.ds(j * _IDX_MINOR, _IDX_MINOR)],
                    sem_g,
                )
                for j in range(_K)
            ]
            for c in copies:
                c.wait()
            pltpu.async_copy(rows_v[b], out_slice(g), sem_o[b])

        pltpu.async_copy(idx_slice(0), idx_v[0], sem_idx)
        step(0, 0, False)
        step(1, 1, False)

        def pair(gp, carry):
            step(gp * 2, 0, True)
            step(gp * 2 + 1, 1, True)
            return carry

        lax.fori_loop(1, ch_per_w // 2, pair, 0)

        pltpu.make_async_copy(idx_slice(ch_per_w - 1), idx_v[0],
                              sem_idx).wait()
        pltpu.make_async_copy(rows_v[0], out_slice(ch_per_w - 2),
                              sem_o[0]).wait()
        pltpu.make_async_copy(rows_v[1], out_slice(ch_per_w - 1),
                              sem_o[1]).wait()

    return k


_gather = _make_kernel(_N)


@jax.jit
def kernel(x, weight):
    idx = x.astype(jnp.int32).reshape(_N // _IDX_MINOR, _IDX_MINOR)
    out = _gather(idx, weight)
    return out.reshape(B, L, DIM)
